# baseline (speedup 1.0000x reference)
import jax
import jax.numpy as jnp
from jax import lax
from jax.experimental import pallas as pl
from jax.experimental.pallas import tpu as pltpu
from jax.experimental.pallas import tpu_sc as plsc

_N_FREQ = 1024
_N_TIME = 2048
_CHUNK = 128
_NCHUNK = _N_FREQ // _CHUNK

_NC = 2
_NS = 16
_NW = _NC * _NS

_TB = 32
_NBUF = 3


def _dispersion_curve():
    freq_indices = jnp.linspace(0.0, 1.0, _N_FREQ)
    freq_ghz = 1.0 + freq_indices * 0.5
    d = 1.0 / freq_ghz ** 2 - 1.0 / jnp.max(freq_ghz) ** 2
    d = d / (jnp.max(d) + 1e-08)
    return d * (_N_TIME * 0.2)


def _splat(v):
    return jnp.full((16,), v, jnp.int32)


def _dedisperse_sc(x3, shifts, batch, n_pol, n_dm):
    t_rows_out = batch * n_dm * n_pol * _N_TIME
    t_per_w = t_rows_out // _NW
    nblk = t_per_w // _TB
    n_shift = batch * n_dm * _NCHUNK

    mesh = plsc.VectorSubcoreMesh(core_axis_name="c", subcore_axis_name="s")

    def body(x_hbm, sh_hbm, out_hbm, sh_v, idx_v, buf_v, sem_g, sem_w):
        wid = lax.axis_index("s") * _NC + lax.axis_index("c")
        pltpu.sync_copy(sh_hbm, sh_v)
        out_base = wid * t_per_w
        page = out_base // _N_TIME
        p_ = lax.rem(page, n_pol)
        bd = page // n_pol
        b_ = bd // n_dm
        in_base = (b_ * n_pol + p_) * _N_TIME
        s_base = bd * _NCHUNK
        t_base = lax.rem(out_base, _N_TIME)

        lane = lax.iota(jnp.int32, 16)
        tmask = _splat(_N_TIME - 1)
        in_base_v = _splat(in_base)
        u = []
        for c in range(_NCHUNK):
            s_c = plsc.load_gather(sh_v, [_splat(s_base + c)])
            u.append([_splat(t_base + h * 16) + lane + s_c
                      for h in range(_TB // 16)])

        def start_gathers(kk):
            slot = kk % _NBUF
            off = _splat(kk * _TB)
            for c in range(_NCHUNK):
                for h in range(_TB // 16):
                    tin = (u[c][h] + off) & tmask
                    idx_v[slot, c, pl.ds(h * 16, 16)] = in_base_v + tin
            for c in range(_NCHUNK):
                pltpu.async_copy(
                    x_hbm.at[idx_v.at[slot, c], pl.ds(c * _CHUNK, _CHUNK)],
                    buf_v.at[slot].at[:, pl.ds(c * _CHUNK, _CHUNK)],
                    sem_g[slot])

        def wait_gathers(kk):
            slot = kk % _NBUF
            for c in range(_NCHUNK):
                pltpu.make_async_copy(
                    x_hbm.at[idx_v.at[slot, c], pl.ds(c * _CHUNK, _CHUNK)],
                    buf_v.at[slot].at[:, pl.ds(c * _CHUNK, _CHUNK)],
                    sem_g[slot]).wait()

        def write_desc(kk):
            slot = kk % _NBUF
            return pltpu.make_async_copy(
                buf_v.at[slot],
                out_hbm.at[pl.ds(out_base + kk * _TB, _TB)],
                sem_w[slot])

        start_gathers(0)
        start_gathers(1)
        for kk in range(nblk):
            wait_gathers(kk)
            write_desc(kk).start()
            if kk >= 1:
                write_desc(kk - 1).wait()
            if kk + 2 < nblk:
                start_gathers(kk + 2)
        write_desc(nblk - 1).wait()

    f = pl.kernel(
        body,
        out_type=jax.ShapeDtypeStruct((t_rows_out, _N_FREQ), jnp.float32),
        mesh=mesh,
        compiler_params=pltpu.CompilerParams(needs_layout_passes=False),
        scratch_types=[
            pltpu.VMEM((n_shift,), jnp.int32),
            pltpu.VMEM((_NBUF, _NCHUNK, _TB), jnp.int32),
            pltpu.VMEM((_NBUF, _TB, _N_FREQ), jnp.float32),
            [pltpu.SemaphoreType.DMA] * _NBUF,
            [pltpu.SemaphoreType.DMA] * _NBUF,
        ],
    )
    return f(x3, shifts)


def kernel(x, dm_values):
    batch, n_pol, n_time, n_freq = x.shape
    n_dm = dm_values.shape[1]
    disp = _dispersion_curve()
    delays = dm_values[:, :, None] * disp[None, None, :]

    shifts = []
    for b in range(batch):
        for d in range(n_dm):
            sample_delays = delays[b, d]
            for fs in range(0, n_freq, _CHUNK):
                avg = sample_delays[fs:fs + _CHUNK].mean().astype(jnp.int32)
                eff = jnp.where(avg > 0, avg, 0)
                shifts.append(lax.rem(eff, jnp.int32(n_time)))
    shifts = jnp.stack(shifts)

    x3 = x.reshape(batch * n_pol * n_time, n_freq)
    out3 = _dedisperse_sc(x3, shifts, batch, n_pol, n_dm)
    out = out3.reshape(batch, n_dm, n_pol, n_time, n_freq)
    return (out, delays)


# speedup vs baseline: 1.4607x; 1.4607x over previous
import jax
import jax.numpy as jnp
from jax import lax
from jax.experimental import pallas as pl
from jax.experimental.pallas import tpu as pltpu
from jax.experimental.pallas import tpu_sc as plsc

_N_FREQ = 1024
_N_TIME = 2048
_CHUNK = 128
_NCHUNK = _N_FREQ // _CHUNK

_NC = 2
_NS = 16
_NW = _NC * _NS

_TB = 32
_NBUF = 2


def _dispersion_curve():
    freq_indices = jnp.linspace(0.0, 1.0, _N_FREQ)
    freq_ghz = 1.0 + freq_indices * 0.5
    d = 1.0 / freq_ghz ** 2 - 1.0 / jnp.max(freq_ghz) ** 2
    d = d / (jnp.max(d) + 1e-08)
    return d * (_N_TIME * 0.2)


def _splat(v):
    return jnp.full((16,), v, jnp.int32)


def _dedisperse_sc(x3, shifts, batch, n_pol, n_dm):
    t_rows_out = batch * n_dm * n_pol * _N_TIME
    t_per_w = t_rows_out // _NW
    nblk = t_per_w // _TB
    n_shift = batch * n_dm * _NCHUNK

    mesh = plsc.VectorSubcoreMesh(core_axis_name="c", subcore_axis_name="s")

    def body(x_hbm, sh_hbm, out_hbm, sh_v, idx_v, buf_v, sem_g):
        wid = lax.axis_index("s") * _NC + lax.axis_index("c")
        pltpu.sync_copy(sh_hbm, sh_v)
        out_base = wid * t_per_w
        page = out_base // _N_TIME
        p_ = lax.rem(page, n_pol)
        bd = page // n_pol
        b_ = bd // n_dm
        in_base = (b_ * n_pol + p_) * _N_TIME
        s_base = bd * _NCHUNK
        t_base = lax.rem(out_base, _N_TIME)

        lane = lax.iota(jnp.int32, 16)
        tmask = _splat(_N_TIME - 1)
        in_base_v = _splat(in_base)
        u = []
        for c in range(_NCHUNK):
            s_c = plsc.load_gather(sh_v, [_splat(s_base + c)])
            u.append([_splat(t_base + h * 16) + lane + s_c
                      for h in range(_TB // 16)])

        def start_gathers(slot, kk):
            off = _splat(kk * _TB)
            for c in range(_NCHUNK):
                for h in range(_TB // 16):
                    tin = (u[c][h] + off) & tmask
                    idx_v[slot, c, pl.ds(h * 16, 16)] = in_base_v + tin
            for c in range(_NCHUNK):
                pltpu.async_copy(
                    x_hbm.at[idx_v.at[slot, c], pl.ds(c * _CHUNK, _CHUNK)],
                    buf_v.at[slot].at[:, pl.ds(c * _CHUNK, _CHUNK)],
                    sem_g[slot])

        def wait_gathers(slot):
            for c in range(_NCHUNK):
                pltpu.make_async_copy(
                    x_hbm.at[idx_v.at[slot, c], pl.ds(c * _CHUNK, _CHUNK)],
                    buf_v.at[slot].at[:, pl.ds(c * _CHUNK, _CHUNK)],
                    sem_g[slot]).wait()

        def write(slot, kk):
            pltpu.sync_copy(
                buf_v.at[slot],
                out_hbm.at[pl.ds(out_base + kk * _TB, _TB)])

        start_gathers(0, 0)

        def grp(k2, carry):
            k0 = k2 * 2
            start_gathers(1, k0 + 1)
            wait_gathers(0)
            write(0, k0)
            start_gathers(0, k0 + 2)
            wait_gathers(1)
            write(1, k0 + 1)
            return carry

        lax.fori_loop(0, nblk // 2 - 1, grp, 0)
        start_gathers(1, nblk - 1)
        wait_gathers(0)
        write(0, nblk - 2)
        wait_gathers(1)
        write(1, nblk - 1)

    f = pl.kernel(
        body,
        out_type=jax.ShapeDtypeStruct((t_rows_out, _N_FREQ), jnp.float32),
        mesh=mesh,
        compiler_params=pltpu.CompilerParams(needs_layout_passes=False),
        scratch_types=[
            pltpu.VMEM((n_shift,), jnp.int32),
            pltpu.VMEM((_NBUF, _NCHUNK, _TB), jnp.int32),
            pltpu.VMEM((_NBUF, _TB, _N_FREQ), jnp.float32),
            [pltpu.SemaphoreType.DMA] * _NBUF,
        ],
    )
    return f(x3, shifts)


def kernel(x, dm_values):
    batch, n_pol, n_time, n_freq = x.shape
    n_dm = dm_values.shape[1]
    disp = _dispersion_curve()
    delays = dm_values[:, :, None] * disp[None, None, :]

    avg = (delays.reshape(batch, n_dm, _NCHUNK, _CHUNK).mean(-1)
           .astype(jnp.int32).reshape(-1))
    shifts = lax.rem(jnp.maximum(avg, 0), jnp.int32(n_time))

    x3 = x.reshape(batch * n_pol * n_time, n_freq)
    out3 = _dedisperse_sc(x3, shifts, batch, n_pol, n_dm)
    out = out3.reshape(batch, n_dm, n_pol, n_time, n_freq)
    return (out, delays)
